# Initial kernel scaffold; baseline (speedup 1.0000x reference)
import jax
import jax.numpy as jnp
from jax.experimental import pallas as pl


def kernel(x, edge_index, W1, b1, W2, b2, fc1_w, fc1_b, fc2_w, fc2_b):
    raise NotImplementedError("write your pallas kernel here")


# speedup vs baseline: 7.8863x; 7.8863x over previous
import functools

import jax
import jax.numpy as jnp
from jax import lax
from jax.experimental import pallas as pl
from jax.experimental.pallas import tpu as pltpu
from jax.experimental.pallas import tpu_sc as plsc

N_NODES = 10000
N_EDGES = 160000
D = 256
DH = 128
DEGW = 16
TILE_R0 = 624
TILE_ROWS = 640
ZR = 128

_mesh = plsc.VectorSubcoreMesh(core_axis_name="c", subcore_axis_name="s")

_f32 = jnp.float32


DK = 40


@functools.partial(
    pl.kernel,
    mesh=_mesh,
    out_type=[
        jax.ShapeDtypeStruct((N_NODES, DEGW), _f32),
        jax.ShapeDtypeStruct((N_NODES, DEGW), _f32),
    ],
    scratch_types=[
        pltpu.VMEM((DK, DEGW), _f32),
        pltpu.VMEM((DK,), jnp.int32),
        pltpu.VMEM((ZR, DEGW), _f32),
        pltpu.VMEM_SHARED((N_NODES, DEGW), _f32),
    ],
)
def _sc_degree(dst_hbm, outa, outb, ones_v, idx_v, zbuf, acc):
    c = lax.axis_index("c")
    s = lax.axis_index("s")

    def fill_ones(i, _):
        ones_v[i, :] = jnp.ones((DEGW,), _f32)
        return 0

    lax.fori_loop(0, DK, fill_ones, 0)

    def fill_zeros(i, _):
        zbuf[i, :] = jnp.zeros((DEGW,), _f32)
        return 0

    lax.fori_loop(0, ZR, fill_zeros, 0)

    row0 = s * TILE_R0
    for m in range(TILE_ROWS // ZR):
        pltpu.sync_copy(zbuf, acc.at[pl.ds(row0 + m * ZR, ZR)])
    plsc.subcore_barrier()

    base0 = c * (N_EDGES // 2) + s * (N_EDGES // 32)

    def chunk(g, _):
        b = base0 + g * DK
        pltpu.sync_copy(dst_hbm.at[pl.ds(b, DK)], idx_v)
        pltpu.sync_copy(ones_v, acc.at[idx_v], add=True)
        return 0

    lax.fori_loop(0, (N_EDGES // 32) // DK, chunk, 0)
    plsc.subcore_barrier()

    @pl.when(c == 0)
    def _():
        pltpu.sync_copy(acc.at[pl.ds(row0, TILE_ROWS)],
                        outa.at[pl.ds(row0, TILE_ROWS)])

    @pl.when(c == 1)
    def _():
        pltpu.sync_copy(acc.at[pl.ds(row0, TILE_ROWS)],
                        outb.at[pl.ds(row0, TILE_ROWS)])


AK = 80


@functools.partial(
    pl.kernel,
    mesh=_mesh,
    out_type=[
        jax.ShapeDtypeStruct((N_NODES, DH), _f32),
        jax.ShapeDtypeStruct((N_NODES, DH), _f32),
    ],
    scratch_types=[
        pltpu.VMEM((AK, DH), _f32),
        pltpu.VMEM((AK,), jnp.int32),
        pltpu.VMEM((AK,), jnp.int32),
        pltpu.VMEM((ZR, DH), _f32),
        pltpu.VMEM_SHARED((N_NODES, DH), _f32),
        pltpu.SemaphoreType.DMA,
    ],
)
def _sc_aggregate(pa_hbm, pb_hbm, src_hbm, dst_hbm, outa, outb,
                  rows_v, src_v, dst_v, zbuf, acc, sem):
    c = lax.axis_index("c")
    s = lax.axis_index("s")

    def fill_zeros(i, _):
        for j in range(DH // 16):
            zbuf[i, pl.ds(j * 16, 16)] = jnp.zeros((16,), _f32)
        return 0

    lax.fori_loop(0, ZR, fill_zeros, 0)

    row0 = s * TILE_R0
    for m in range(TILE_ROWS // ZR):
        pltpu.sync_copy(zbuf, acc.at[pl.ds(row0 + m * ZR, ZR)])
    plsc.subcore_barrier()

    def run(p_hbm, out_hbm):
        base0 = s * (N_EDGES // 16)

        def chunk(g, _):
            b = base0 + g * AK
            pltpu.sync_copy(src_hbm.at[pl.ds(b, AK)], src_v)
            pltpu.sync_copy(dst_hbm.at[pl.ds(b, AK)], dst_v)
            pltpu.async_copy(p_hbm.at[src_v], rows_v, sem).wait()
            pltpu.sync_copy(rows_v, acc.at[dst_v], add=True)
            return 0

        lax.fori_loop(0, (N_EDGES // 16) // AK, chunk, 0)
        plsc.subcore_barrier()
        pltpu.sync_copy(acc.at[pl.ds(row0, TILE_ROWS)],
                        out_hbm.at[pl.ds(row0, TILE_ROWS)])

    @pl.when(c == 0)
    def _():
        run(pa_hbm, outa)

    @pl.when(c == 1)
    def _():
        run(pb_hbm, outb)


BN = 400


def _dinv_from(da, db):
    deg = da[:, 0:1] + db[:, 0:1] + 1.0
    return lax.rsqrt(deg)


def _tc1_body(x_ref, w_ref, da_ref, db_ref, pa_ref, pb_ref):
    dinv = _dinv_from(da_ref[...], db_ref[...])
    m = jnp.dot(x_ref[...], w_ref[...], preferred_element_type=_f32)
    p = m * dinv
    pa_ref[...] = p[:, :DH]
    pb_ref[...] = p[:, DH:]


def _tc2_body(aa_ref, ab_ref, pa_ref, pb_ref, da_ref, db_ref, b1_ref,
              w2_ref, oa_ref, ob_ref):
    dinv = _dinv_from(da_ref[...], db_ref[...])
    hsum = jnp.concatenate(
        [aa_ref[...] + pa_ref[...], ab_ref[...] + pb_ref[...]], axis=1)
    h1 = jax.nn.relu(dinv * hsum + b1_ref[...])
    m2 = jnp.dot(h1, w2_ref[...], preferred_element_type=_f32)
    p2 = m2 * dinv
    oa_ref[...] = p2[:, :DH]
    ob_ref[...] = p2[:, DH:]


def _tc3_body(aa_ref, ab_ref, pa_ref, pb_ref, da_ref, db_ref, b2_ref,
              f1w_ref, f1b_ref, f2w_ref, f2b_ref, out_ref):
    dinv = _dinv_from(da_ref[...], db_ref[...])
    hsum = jnp.concatenate(
        [aa_ref[...] + pa_ref[...], ab_ref[...] + pb_ref[...]], axis=1)
    h2 = dinv * hsum + b2_ref[...]
    row = lax.broadcasted_iota(jnp.int32, (16, 128), 0)
    col = lax.broadcasted_iota(jnp.int32, (16, 128), 1)
    g = jnp.where(col // 8 == row, 0.125, 0.0).astype(_f32)
    agg = jnp.dot(g, h2, preferred_element_type=_f32)
    y = jax.nn.relu(jnp.dot(agg, f1w_ref[...], preferred_element_type=_f32)
                    + f1b_ref[...])
    out_ref[...] = (jnp.dot(y, f2w_ref[...], preferred_element_type=_f32)
                    + f2b_ref[...])


def _node_spec(w):
    return pl.BlockSpec((BN, w), lambda i: (i, 0))


def _full_spec(shape):
    return pl.BlockSpec(shape, lambda i: (0,) * len(shape))


_tc1 = pl.pallas_call(
    _tc1_body,
    grid=(N_NODES // BN,),
    in_specs=[_node_spec(D), _full_spec((D, D)),
              _node_spec(DEGW), _node_spec(DEGW)],
    out_specs=[_node_spec(DH), _node_spec(DH)],
    out_shape=[jax.ShapeDtypeStruct((N_NODES, DH), _f32)] * 2,
)

_tc2 = pl.pallas_call(
    _tc2_body,
    grid=(N_NODES // BN,),
    in_specs=[_node_spec(DH)] * 4 + [_node_spec(DEGW)] * 2
    + [_full_spec((1, D)), _full_spec((D, D))],
    out_specs=[_node_spec(DH), _node_spec(DH)],
    out_shape=[jax.ShapeDtypeStruct((N_NODES, DH), _f32)] * 2,
)

_tc3 = pl.pallas_call(
    _tc3_body,
    grid=(1,),
    in_specs=[_full_spec((128, DH))] * 4 + [_full_spec((128, DEGW))] * 2
    + [_full_spec((1, D)), _full_spec((D, D)), _full_spec((1, D)),
       _full_spec((D, 64)), _full_spec((1, 64))],
    out_specs=[_full_spec((16, 64))],
    out_shape=[jax.ShapeDtypeStruct((16, 64), _f32)],
)


def kernel(x, edge_index, W1, b1, W2, b2, fc1_w, fc1_b, fc2_w, fc2_b):
    src = edge_index[0]
    dst = edge_index[1]
    dega, degb = _sc_degree(dst)
    p1a, p1b = _tc1(x, W1, dega, degb)
    a1a, a1b = _sc_aggregate(p1a, p1b, src, dst)
    p2a, p2b = _tc2(a1a, a1b, p1a, p1b, dega, degb,
                    b1.reshape(1, D), W2)
    a2a, a2b = _sc_aggregate(p2a, p2b, src, dst)
    (y,) = _tc3(a2a, a2b, p2a, p2b, dega, degb, b2.reshape(1, D),
                fc1_w, fc1_b.reshape(1, D), fc2_w, fc2_b.reshape(1, 64))
    return y


# speedup vs baseline: 16.5109x; 2.0936x over previous
import functools

import jax
import jax.numpy as jnp
from jax import lax
from jax.experimental import pallas as pl
from jax.experimental.pallas import tpu as pltpu
from jax.experimental.pallas import tpu_sc as plsc

N_NODES = 10000
N_EDGES = 160000
D = 256
DH = 128
DEGW = 16
TILE_R0 = 624
TILE_ROWS = 640
ZR = 128

_mesh = plsc.VectorSubcoreMesh(core_axis_name="c", subcore_axis_name="s")

_f32 = jnp.float32


DK = 40
DG = (N_EDGES // 32) // DK


@functools.partial(
    pl.kernel,
    mesh=_mesh,
    out_type=[
        jax.ShapeDtypeStruct((N_NODES, DEGW), _f32),
        jax.ShapeDtypeStruct((N_NODES, DEGW), _f32),
    ],
    scratch_types=[
        pltpu.VMEM((DK, DEGW), _f32),
        pltpu.VMEM((DG, DK), jnp.int32),
        pltpu.VMEM((ZR, DEGW), _f32),
        pltpu.VMEM_SHARED((N_NODES, DEGW), _f32),
    ],
)
def _sc_degree(dst_hbm, outa, outb, ones_v, idx_v, zbuf, acc):
    c = lax.axis_index("c")
    s = lax.axis_index("s")

    def fill_ones(i, _):
        ones_v[i, :] = jnp.ones((DEGW,), _f32)
        return 0

    lax.fori_loop(0, DK, fill_ones, 0)

    def fill_zeros(i, _):
        zbuf[i, :] = jnp.zeros((DEGW,), _f32)
        return 0

    lax.fori_loop(0, ZR, fill_zeros, 0)

    pltpu.sync_copy(dst_hbm.at[c * 16 + s], idx_v)

    row0 = s * TILE_R0
    for m in range(TILE_ROWS // ZR):
        pltpu.sync_copy(zbuf, acc.at[pl.ds(row0 + m * ZR, ZR)])
    plsc.subcore_barrier()

    def chunk(g, _):
        pltpu.sync_copy(ones_v, acc.at[idx_v.at[g]], add=True)
        return 0

    lax.fori_loop(0, DG, chunk, 0)
    plsc.subcore_barrier()

    @pl.when(c == 0)
    def _():
        pltpu.sync_copy(acc.at[pl.ds(row0, TILE_ROWS)],
                        outa.at[pl.ds(row0, TILE_ROWS)])

    @pl.when(c == 1)
    def _():
        pltpu.sync_copy(acc.at[pl.ds(row0, TILE_ROWS)],
                        outb.at[pl.ds(row0, TILE_ROWS)])


AK = 100
AG = (N_EDGES // 16) // AK
AZR = 16


@functools.partial(
    pl.kernel,
    mesh=_mesh,
    out_type=[
        jax.ShapeDtypeStruct((N_NODES, DH), _f32),
        jax.ShapeDtypeStruct((N_NODES, DH), _f32),
    ],
    scratch_types=[
        pltpu.VMEM((2, AK, DH), _f32),
        pltpu.VMEM((2, 2, AK), jnp.int32),
        pltpu.VMEM((AZR, DH), _f32),
        pltpu.VMEM_SHARED((N_NODES, DH), _f32),
        pltpu.SemaphoreType.DMA,
        pltpu.SemaphoreType.DMA,
        pltpu.SemaphoreType.DMA,
        pltpu.SemaphoreType.DMA,
    ],
)
def _sc_aggregate(pa_hbm, pb_hbm, ei_hbm, outa, outb,
                  rows_v, idx_v, zbuf, acc, gsem0, gsem1, isem0, isem1):
    c = lax.axis_index("c")
    s = lax.axis_index("s")

    def fill_zeros(i, _):
        for j in range(DH // 16):
            zbuf[i, pl.ds(j * 16, 16)] = jnp.zeros((16,), _f32)
        return 0

    lax.fori_loop(0, AZR, fill_zeros, 0)

    row0 = s * TILE_R0
    for m in range(TILE_ROWS // AZR):
        pltpu.sync_copy(zbuf, acc.at[pl.ds(row0 + m * AZR, AZR)])
    plsc.subcore_barrier()

    gsems = (gsem0, gsem1)
    isems = (isem0, isem1)

    def idxcpy(g, b):
        return pltpu.make_async_copy(ei_hbm.at[s, g], idx_v.at[b], isems[b])

    def run(p_hbm, out_hbm):
        def gather(g, b):
            return pltpu.make_async_copy(
                p_hbm.at[idx_v.at[b].at[0]], rows_v.at[b], gsems[b])

        def scat(b, g):
            pltpu.sync_copy(rows_v.at[b], acc.at[idx_v.at[b].at[1]],
                            add=True)

        idxcpy(0, 0).start()
        idxcpy(1, 1).start()
        idxcpy(0, 0).wait()
        gather(0, 0).start()

        def pipe(i, _):
            g0 = 2 * i
            idxcpy(g0 + 1, 1).wait()
            gather(g0 + 1, 1).start()
            gather(g0, 0).wait()
            scat(0, g0)

            @pl.when(g0 + 2 < AG)
            def _():
                idxcpy(g0 + 2, 0).start()
                idxcpy(g0 + 2, 0).wait()
                gather(g0 + 2, 0).start()

            gather(g0 + 1, 1).wait()
            scat(1, g0 + 1)

            @pl.when(g0 + 3 < AG)
            def _():
                idxcpy(g0 + 3, 1).start()

            return 0

        lax.fori_loop(0, AG // 2, pipe, 0)
        plsc.subcore_barrier()
        pltpu.sync_copy(acc.at[pl.ds(row0, TILE_ROWS)],
                        out_hbm.at[pl.ds(row0, TILE_ROWS)])

    @pl.when(c == 0)
    def _():
        run(pa_hbm, outa)

    @pl.when(c == 1)
    def _():
        run(pb_hbm, outb)


BN = 400


def _dinv_from(da, db):
    deg = da[:, 0:1] + db[:, 0:1] + 1.0
    return lax.rsqrt(deg)


def _tc1_body(x_ref, w_ref, da_ref, db_ref, pa_ref, pb_ref):
    dinv = _dinv_from(da_ref[...], db_ref[...])
    m = jnp.dot(x_ref[...], w_ref[...], preferred_element_type=_f32)
    p = m * dinv
    pa_ref[...] = p[:, :DH]
    pb_ref[...] = p[:, DH:]


def _tc2_body(aa_ref, ab_ref, pa_ref, pb_ref, da_ref, db_ref, b1_ref,
              w2_ref, oa_ref, ob_ref):
    dinv = _dinv_from(da_ref[...], db_ref[...])
    hsum = jnp.concatenate(
        [aa_ref[...] + pa_ref[...], ab_ref[...] + pb_ref[...]], axis=1)
    h1 = jax.nn.relu(dinv * hsum + b1_ref[...])
    m2 = jnp.dot(h1, w2_ref[...], preferred_element_type=_f32)
    p2 = m2 * dinv
    oa_ref[...] = p2[:, :DH]
    ob_ref[...] = p2[:, DH:]


def _tc3_body(aa_ref, ab_ref, pa_ref, pb_ref, da_ref, db_ref, b2_ref,
              f1w_ref, f1b_ref, f2w_ref, f2b_ref, out_ref):
    dinv = _dinv_from(da_ref[...], db_ref[...])
    hsum = jnp.concatenate(
        [aa_ref[...] + pa_ref[...], ab_ref[...] + pb_ref[...]], axis=1)
    h2 = dinv * hsum + b2_ref[...]
    row = lax.broadcasted_iota(jnp.int32, (16, 128), 0)
    col = lax.broadcasted_iota(jnp.int32, (16, 128), 1)
    g = jnp.where(col // 8 == row, 0.125, 0.0).astype(_f32)
    agg = jnp.dot(g, h2, preferred_element_type=_f32)
    y = jax.nn.relu(jnp.dot(agg, f1w_ref[...], preferred_element_type=_f32)
                    + f1b_ref[...])
    out_ref[...] = (jnp.dot(y, f2w_ref[...], preferred_element_type=_f32)
                    + f2b_ref[...])


def _node_spec(w):
    return pl.BlockSpec((BN, w), lambda i: (i, 0))


def _full_spec(shape):
    return pl.BlockSpec(shape, lambda i: (0,) * len(shape))


_tc1 = pl.pallas_call(
    _tc1_body,
    grid=(N_NODES // BN,),
    in_specs=[_node_spec(D), _full_spec((D, D)),
              _node_spec(DEGW), _node_spec(DEGW)],
    out_specs=[_node_spec(DH), _node_spec(DH)],
    out_shape=[jax.ShapeDtypeStruct((N_NODES, DH), _f32)] * 2,
)

_tc2 = pl.pallas_call(
    _tc2_body,
    grid=(N_NODES // BN,),
    in_specs=[_node_spec(DH)] * 4 + [_node_spec(DEGW)] * 2
    + [_full_spec((1, D)), _full_spec((D, D))],
    out_specs=[_node_spec(DH), _node_spec(DH)],
    out_shape=[jax.ShapeDtypeStruct((N_NODES, DH), _f32)] * 2,
)

_tc3 = pl.pallas_call(
    _tc3_body,
    grid=(1,),
    in_specs=[_full_spec((128, DH))] * 4 + [_full_spec((128, DEGW))] * 2
    + [_full_spec((1, D)), _full_spec((D, D)), _full_spec((1, D)),
       _full_spec((D, 64)), _full_spec((1, 64))],
    out_specs=[_full_spec((16, 64))],
    out_shape=[jax.ShapeDtypeStruct((16, 64), _f32)],
)


def kernel(x, edge_index, W1, b1, W2, b2, fc1_w, fc1_b, fc2_w, fc2_b):
    src = edge_index[0]
    dst = edge_index[1]
    ei2 = jnp.stack(
        [src.reshape(16, AG, AK), dst.reshape(16, AG, AK)], axis=2)
    dega, degb = _sc_degree(dst.reshape(32, DG, DK))
    p1a, p1b = _tc1(x, W1, dega, degb)
    a1a, a1b = _sc_aggregate(p1a, p1b, ei2)
    p2a, p2b = _tc2(a1a, a1b, p1a, p1b, dega, degb,
                    b1.reshape(1, D), W2)
    a2a, a2b = _sc_aggregate(p2a, p2b, ei2)
    (y,) = _tc3(a2a, a2b, p2a, p2b, dega, degb, b2.reshape(1, D),
                fc1_w, fc1_b.reshape(1, D), fc2_w, fc2_b.reshape(1, 64))
    return y
